# baseline (speedup 1.0000x reference)
import jax
import jax.numpy as jnp
from jax.experimental import pallas as pl
from jax.experimental.pallas import tpu as pltpu


def _round_up(x, m):
    return ((x + m - 1) // m) * m


def _edge_kernel(edges_ref, ht_ref, wt_ref, b_ref, out_ref, g_ref):
    te = out_ref.shape[0]
    i = pl.program_id(0)
    slot = jax.lax.rem(i, 2)
    prev = 1 - slot

    z = jnp.dot(wt_ref[...], g_ref[prev],
                preferred_element_type=jnp.float32) + b_ref[...]
    m = jnp.max(z, axis=0, keepdims=True)
    e = jnp.exp(z - m)
    p = e / jnp.sum(e, axis=0, keepdims=True)
    out_ref[...] = p.T

    idx = edges_ref[...]
    ht = ht_ref[...]
    n_nodes = ht.shape[1]
    nsplit = 16
    w = te // nsplit
    iota16 = jax.lax.broadcasted_iota(jnp.int16, (n_nodes, 2 * w), 0)
    for part in range(nsplit):
        sl = slice(part * w, (part + 1) * w)
        idx_cat = jnp.concatenate([idx[0:1, sl], idx[1:2, sl]], axis=1)
        oh = jnp.where(iota16 == idx_cat.astype(jnp.int16),
                       jnp.bfloat16(1), jnp.bfloat16(0))
        g = jnp.dot(ht, oh, preferred_element_type=jnp.float32)
        embt = g[:, :w] * g[:, w:]
        g_ref[slot, :, sl] = embt.astype(jnp.bfloat16)


def _fuse_affine(weights, biases):
    w = weights[0].T
    b = biases[0]
    for wi, bi in zip(weights[1:], biases[1:]):
        w = w @ wi.T
        b = b @ wi.T + bi
    return w, b


def kernel(h, edges, w1, b1, w2, b2, *, tile_e=32768):
    n_nodes, nin = h.shape
    e_total = edges.shape[0]

    w_fused, b_fused = _fuse_affine([w1, w2], [b1, b2])
    nclass = w_fused.shape[1]
    wt = w_fused.T.astype(jnp.bfloat16)
    b_col = b_fused.reshape(nclass, 1).astype(jnp.float32)

    ht = h.T.astype(jnp.bfloat16)

    half = _round_up(max((e_total + 1) // 2, 128), 128)
    tile_e = max(128, min(_round_up(tile_e, 128), half))
    e_pad = _round_up(e_total, tile_e)
    n_tiles = e_pad // tile_e
    grid = (n_tiles + 1,)

    edges_t = jnp.pad(edges.astype(jnp.int32).T, ((0, 0), (0, e_pad - e_total)))

    cost = pl.CostEstimate(
        flops=int(2 * nin * n_nodes * 2 * e_pad
                  + e_pad * nin
                  + 2 * nclass * nin * e_pad
                  + 4 * e_pad * nclass),
        transcendentals=int(e_pad * nclass),
        bytes_accessed=int(4 * (2 * e_pad + nclass + nclass * e_pad)
                           + 2 * (nin * n_nodes + nclass * nin)),
    )
    out_t = pl.pallas_call(
        _edge_kernel,
        out_shape=jax.ShapeDtypeStruct((e_pad, nclass), jnp.float32),
        grid=grid,
        in_specs=[
            pl.BlockSpec((2, tile_e),
                         lambda i: (0, jnp.minimum(i, n_tiles - 1))),
            pl.BlockSpec((nin, n_nodes), lambda i: (0, 0)),
            pl.BlockSpec((nclass, nin), lambda i: (0, 0)),
            pl.BlockSpec((nclass, 1), lambda i: (0, 0)),
        ],
        out_specs=pl.BlockSpec((tile_e, nclass),
                               lambda i: (jnp.maximum(i - 1, 0), 0)),
        scratch_shapes=[pltpu.VMEM((2, nin, tile_e), jnp.bfloat16)],
        compiler_params=pltpu.CompilerParams(
            dimension_semantics=("arbitrary",),
            vmem_limit_bytes=64 * 1024 * 1024),
        cost_estimate=cost,
    )(edges_t, ht, wt, b_col)

    return out_t[:e_total]


# speedup vs baseline: 1.7940x; 1.7940x over previous
import jax
import jax.numpy as jnp
from jax.experimental import pallas as pl
from jax.experimental.pallas import tpu as pltpu


def _round_up(x, m):
    return ((x + m - 1) // m) * m


def _edge_kernel(edges_ref, ht_ref, wt_ref, b_ref, out_ref, g_ref):
    te = out_ref.shape[1]
    i = pl.program_id(0)
    slot = jax.lax.rem(i, 2)
    prev = 1 - slot

    z = jnp.dot(wt_ref[...], g_ref[prev],
                preferred_element_type=jnp.float32) + b_ref[...]
    m = jnp.max(z, axis=0, keepdims=True)
    e = jnp.exp(z - m)
    out_ref[...] = e / jnp.sum(e, axis=0, keepdims=True)

    idx = edges_ref[...]
    ht = ht_ref[...]
    n_nodes = ht.shape[1]
    nsplit = 32
    w = te // nsplit
    iota16 = jax.lax.broadcasted_iota(jnp.int16, (n_nodes, 2 * w), 0)
    for part in range(nsplit):
        sl = slice(part * w, (part + 1) * w)
        idx_cat = jnp.concatenate([idx[0:1, sl], idx[1:2, sl]], axis=1)
        oh = jnp.where(iota16 == idx_cat.astype(jnp.int16),
                       jnp.bfloat16(1), jnp.bfloat16(0))
        g = jnp.dot(ht, oh, preferred_element_type=jnp.float32)
        embt = g[:, :w] * g[:, w:]
        g_ref[slot, :, sl] = embt.astype(jnp.bfloat16)


def _fuse_affine(weights, biases):
    w = weights[0].T
    b = biases[0]
    for wi, bi in zip(weights[1:], biases[1:]):
        w = w @ wi.T
        b = b @ wi.T + bi
    return w, b


def kernel(h, edges, w1, b1, w2, b2, *, tile_e=65536):
    n_nodes, nin = h.shape
    e_total = edges.shape[0]

    w_fused, b_fused = _fuse_affine([w1, w2], [b1, b2])
    nclass = w_fused.shape[1]
    wt = w_fused.T.astype(jnp.bfloat16)
    b_col = b_fused.reshape(nclass, 1).astype(jnp.float32)

    ht = h.T.astype(jnp.bfloat16)

    half = _round_up(max((e_total + 1) // 2, 128), 128)
    tile_e = max(128, min(_round_up(tile_e, 128), half))
    e_pad = _round_up(e_total, tile_e)
    n_tiles = e_pad // tile_e
    grid = (n_tiles + 1,)

    edges_t = jnp.pad(edges.astype(jnp.int32).T, ((0, 0), (0, e_pad - e_total)))

    cost = pl.CostEstimate(
        flops=int(2 * nin * n_nodes * 2 * e_pad
                  + e_pad * nin
                  + 2 * nclass * nin * e_pad
                  + 4 * e_pad * nclass),
        transcendentals=int(e_pad * nclass),
        bytes_accessed=int(4 * (2 * e_pad + nclass + nclass * e_pad)
                           + 2 * (nin * n_nodes + nclass * nin)),
    )
    out_t = pl.pallas_call(
        _edge_kernel,
        out_shape=jax.ShapeDtypeStruct((nclass, e_pad), jnp.float32),
        grid=grid,
        in_specs=[
            pl.BlockSpec((2, tile_e),
                         lambda i: (0, jnp.minimum(i, n_tiles - 1))),
            pl.BlockSpec((nin, n_nodes), lambda i: (0, 0)),
            pl.BlockSpec((nclass, nin), lambda i: (0, 0)),
            pl.BlockSpec((nclass, 1), lambda i: (0, 0)),
        ],
        out_specs=pl.BlockSpec((nclass, tile_e),
                               lambda i: (0, jnp.maximum(i - 1, 0))),
        scratch_shapes=[pltpu.VMEM((2, nin, tile_e), jnp.bfloat16)],
        compiler_params=pltpu.CompilerParams(
            dimension_semantics=("arbitrary",),
            vmem_limit_bytes=64 * 1024 * 1024),
        cost_estimate=cost,
    )(edges_t, ht, wt, b_col)

    return out_t.T[:e_total]


# speedup vs baseline: 1.7989x; 1.0028x over previous
import jax
import jax.numpy as jnp
from jax.experimental import pallas as pl
from jax.experimental.pallas import tpu as pltpu


def _round_up(x, m):
    return ((x + m - 1) // m) * m


def _edge_kernel(edges_ref, ht_ref, wt_ref, b_ref, out_ref, g_ref):
    te = out_ref.shape[1]
    i = pl.program_id(0)
    slot = jax.lax.rem(i, 2)
    prev = 1 - slot

    z = jnp.dot(wt_ref[...], g_ref[prev],
                preferred_element_type=jnp.float32) + b_ref[...]
    m = jnp.max(z, axis=0, keepdims=True)
    e = jnp.exp(z - m)
    out_ref[...] = e / jnp.sum(e, axis=0, keepdims=True)

    idx = edges_ref[...]
    ht = ht_ref[...]
    n_nodes = ht.shape[1]
    nsplit = 16
    w = te // nsplit
    iota16 = jax.lax.broadcasted_iota(jnp.int16, (n_nodes, 2 * w), 0)
    for part in range(nsplit):
        sl = slice(part * w, (part + 1) * w)
        idx_cat = jnp.concatenate([idx[0:1, sl], idx[1:2, sl]], axis=1)
        oh = jnp.where(iota16 == idx_cat.astype(jnp.int16),
                       jnp.bfloat16(1), jnp.bfloat16(0))
        g = jnp.dot(ht, oh, preferred_element_type=jnp.float32)
        embt = g[:, :w] * g[:, w:]
        g_ref[slot, :, sl] = embt.astype(jnp.bfloat16)


def _fuse_affine(weights, biases):
    w = weights[0].T
    b = biases[0]
    for wi, bi in zip(weights[1:], biases[1:]):
        w = w @ wi.T
        b = b @ wi.T + bi
    return w, b


def kernel(h, edges, w1, b1, w2, b2, *, tile_e=32768):
    n_nodes, nin = h.shape
    e_total = edges.shape[0]

    w_fused, b_fused = _fuse_affine([w1, w2], [b1, b2])
    nclass = w_fused.shape[1]
    wt = w_fused.T.astype(jnp.bfloat16)
    b_col = b_fused.reshape(nclass, 1).astype(jnp.float32)

    ht = h.T.astype(jnp.bfloat16)

    half = _round_up(max((e_total + 1) // 2, 128), 128)
    tile_e = max(128, min(_round_up(tile_e, 128), half))
    e_pad = _round_up(e_total, tile_e)
    n_tiles = e_pad // tile_e
    grid = (n_tiles + 1,)

    edges_t = jnp.pad(edges.astype(jnp.int32).T, ((0, 0), (0, e_pad - e_total)))

    cost = pl.CostEstimate(
        flops=int(2 * nin * n_nodes * 2 * e_pad
                  + e_pad * nin
                  + 2 * nclass * nin * e_pad
                  + 4 * e_pad * nclass),
        transcendentals=int(e_pad * nclass),
        bytes_accessed=int(4 * (2 * e_pad + nclass + nclass * e_pad)
                           + 2 * (nin * n_nodes + nclass * nin)),
    )
    out_t = pl.pallas_call(
        _edge_kernel,
        out_shape=jax.ShapeDtypeStruct((nclass, e_pad), jnp.float32),
        grid=grid,
        in_specs=[
            pl.BlockSpec((2, tile_e),
                         lambda i: (0, jnp.minimum(i, n_tiles - 1))),
            pl.BlockSpec((nin, n_nodes), lambda i: (0, 0)),
            pl.BlockSpec((nclass, nin), lambda i: (0, 0)),
            pl.BlockSpec((nclass, 1), lambda i: (0, 0)),
        ],
        out_specs=pl.BlockSpec((nclass, tile_e),
                               lambda i: (0, jnp.maximum(i - 1, 0))),
        scratch_shapes=[pltpu.VMEM((2, nin, tile_e), jnp.bfloat16)],
        compiler_params=pltpu.CompilerParams(
            dimension_semantics=("arbitrary",),
            vmem_limit_bytes=64 * 1024 * 1024),
        cost_estimate=cost,
    )(edges_t, ht, wt, b_col)

    return out_t.T[:e_total]
